# baseline (speedup 1.0000x reference)
import functools

import jax
import jax.numpy as jnp
from jax import lax
from jax.experimental import pallas as pl
from jax.experimental.pallas import tpu as pltpu
from jax.experimental.pallas import tpu_sc as plsc

NUM_CLASSES = 40
OUT_CHANNELS = 512
N = 100000

NUM_WORKERS = 32
CHUNK = 80
NUM_CHUNKS = N // CHUNK
LANES = 16


@functools.partial(
    pl.kernel,
    mesh=plsc.VectorSubcoreMesh(core_axis_name="c", subcore_axis_name="s"),
    out_type=jax.ShapeDtypeStruct((N, OUT_CHANNELS), jnp.float32),
    scratch_types=[
        pltpu.VMEM((CHUNK,), jnp.int32),
        pltpu.VMEM((CHUNK,), jnp.int32),
        pltpu.VMEM((CHUNK,), jnp.int32),
        pltpu.VMEM((CHUNK, OUT_CHANNELS), jnp.float32),
        pltpu.SemaphoreType.DMA,
    ],
)
def _masked_gather(y_hbm, m_hbm, emb_hbm, out_hbm, y_v, m_v, idx_v, rows_v, sem):
    wid = lax.axis_index("s") * 2 + lax.axis_index("c")
    n_chunks_w = (NUM_CHUNKS - wid + NUM_WORKERS - 1) // NUM_WORKERS

    def body(t, carry):
        c = wid + t * NUM_WORKERS
        base = c * CHUNK
        pltpu.sync_copy(y_hbm.at[pl.ds(base, CHUNK)], y_v)
        pltpu.sync_copy(m_hbm.at[pl.ds(base, CHUNK)], m_v)
        for i in range(CHUNK // LANES):
            sl = pl.ds(i * LANES, LANES)
            idx_v[sl] = jnp.where(m_v[sl] != 0, y_v[sl], NUM_CLASSES)
        pltpu.async_copy(emb_hbm.at[idx_v], rows_v, sem).wait()
        pltpu.sync_copy(rows_v, out_hbm.at[pl.ds(base, CHUNK)])
        return carry

    lax.fori_loop(0, n_chunks_w, body, 0)


def kernel(y, mask, emb):
    y32 = y.astype(jnp.int32)
    m32 = mask.astype(jnp.int32)
    emb2 = jnp.concatenate(
        [emb, jnp.zeros((1, OUT_CHANNELS), emb.dtype)], axis=0)
    return _masked_gather(y32, m32, emb2)


# speedup vs baseline: 1.0588x; 1.0588x over previous
import functools

import jax
import jax.numpy as jnp
from jax import lax
from jax.experimental import pallas as pl
from jax.experimental.pallas import tpu as pltpu
from jax.experimental.pallas import tpu_sc as plsc

NUM_CLASSES = 40
OUT_CHANNELS = 512
N = 100000

NUM_WORKERS = 32
CHUNK = 80
NUM_CHUNKS = N // CHUNK
MAX_T = (NUM_CHUNKS + NUM_WORKERS - 1) // NUM_WORKERS
SPAN = MAX_T * CHUNK
N_PAD = (NUM_CHUNKS - 1) * CHUNK + SPAN
LANES = 16


@functools.partial(
    pl.kernel,
    mesh=plsc.VectorSubcoreMesh(core_axis_name="c", subcore_axis_name="s"),
    out_type=jax.ShapeDtypeStruct((N, OUT_CHANNELS), jnp.float32),
    scratch_types=[
        pltpu.VMEM((SPAN,), jnp.int32),
        pltpu.VMEM((SPAN,), jnp.int32),
        pltpu.VMEM((SPAN,), jnp.int32),
        pltpu.VMEM((CHUNK,), jnp.int32),
        pltpu.VMEM((CHUNK,), jnp.int32),
        pltpu.VMEM((CHUNK, OUT_CHANNELS), jnp.float32),
        pltpu.VMEM((CHUNK, OUT_CHANNELS), jnp.float32),
        pltpu.SemaphoreType.DMA,
        pltpu.SemaphoreType.DMA,
        pltpu.SemaphoreType.DMA,
        pltpu.SemaphoreType.DMA,
    ],
)
def _masked_gather(y_hbm, m_hbm, emb_hbm, out_hbm,
                   y_v, m_v, idx_v, idxc0, idxc1, rows0, rows1,
                   gsem0, gsem1, wsem0, wsem1):
    w = lax.axis_index("s") * 2 + lax.axis_index("c")
    nt = jnp.where(w < 2, MAX_T, MAX_T - 1)
    start_chunk = (MAX_T - 1) * w + jnp.minimum(w, 2)
    base = start_chunk * CHUNK

    pltpu.sync_copy(y_hbm.at[pl.ds(base, SPAN)], y_v)
    pltpu.sync_copy(m_hbm.at[pl.ds(base, SPAN)], m_v)

    def sel_body(i, carry):
        sl = pl.ds(i * LANES, LANES)
        idx_v[sl] = jnp.where(m_v[sl] != 0, y_v[sl], NUM_CLASSES)
        return carry
    lax.fori_loop(0, SPAN // LANES, sel_body, 0)

    idxc = (idxc0, idxc1)
    rows = (rows0, rows1)
    gsem = (gsem0, gsem1)
    wsem = (wsem0, wsem1)

    def gather_desc(t, b):
        return pltpu.make_async_copy(emb_hbm.at[idxc[b]], rows[b], gsem[b])

    def write_desc(t, b):
        dst = out_hbm.at[pl.ds(base + t * CHUNK, CHUNK)]
        return pltpu.make_async_copy(rows[b], dst, wsem[b])

    for t in range(MAX_T + 2):
        b = t % 2
        if t >= 2:
            @pl.when(t - 2 < nt)
            def _(t=t, b=b):
                write_desc(t - 2, b).wait()
        if t < MAX_T:
            @pl.when(t < nt)
            def _(t=t, b=b):
                for i in range(CHUNK // LANES):
                    dst_sl = pl.ds(i * LANES, LANES)
                    idxc[b][dst_sl] = idx_v[pl.ds(t * CHUNK + i * LANES, LANES)]
                gather_desc(t, b).start()
        if t >= 1:
            @pl.when(t - 1 < nt)
            def _(t=t, b=b):
                gather_desc(t - 1, 1 - b).wait()
                write_desc(t - 1, 1 - b).start()


def kernel(y, mask, emb):
    y32 = jnp.pad(y.astype(jnp.int32), (0, N_PAD - N))
    m32 = jnp.pad(mask.astype(jnp.int32), (0, N_PAD - N))
    emb2 = jnp.concatenate(
        [emb, jnp.zeros((1, OUT_CHANNELS), emb.dtype)], axis=0)
    return _masked_gather(y32, m32, emb2)


# speedup vs baseline: 6.3192x; 5.9684x over previous
import functools

import jax
import jax.numpy as jnp
from jax import lax
from jax.experimental import pallas as pl
from jax.experimental.pallas import tpu as pltpu
from jax.experimental.pallas import tpu_sc as plsc

NUM_CLASSES = 40
OUT_CHANNELS = 512
N = 100000

NUM_WORKERS = 32
CHUNK = 80
NUM_CHUNKS = N // CHUNK
MAX_T = (NUM_CHUNKS + NUM_WORKERS - 1) // NUM_WORKERS
SPAN = MAX_T * CHUNK
N_PAD = (NUM_CHUNKS - 1) * CHUNK + SPAN
LANES = 16
VPR = OUT_CHANNELS // LANES


@functools.partial(
    pl.kernel,
    mesh=plsc.VectorSubcoreMesh(core_axis_name="c", subcore_axis_name="s"),
    out_type=jax.ShapeDtypeStruct((N, OUT_CHANNELS), jnp.float32),
    scratch_types=[
        pltpu.VMEM((NUM_CLASSES + 1, OUT_CHANNELS), jnp.float32),
        pltpu.VMEM((SPAN,), jnp.int32),
        pltpu.VMEM((SPAN,), jnp.int32),
        pltpu.VMEM((SPAN,), jnp.int32),
        pltpu.VMEM((CHUNK, OUT_CHANNELS), jnp.float32),
        pltpu.VMEM((CHUNK, OUT_CHANNELS), jnp.float32),
        pltpu.SemaphoreType.DMA,
        pltpu.SemaphoreType.DMA,
    ],
)
def _masked_gather(y_hbm, m_hbm, emb_hbm, out_hbm,
                   table_v, y_v, m_v, idx_v, rows0, rows1, wsem0, wsem1):
    w = lax.axis_index("s") * 2 + lax.axis_index("c")
    nt = jnp.where(w < 2, MAX_T, MAX_T - 1)
    start_chunk = (MAX_T - 1) * w + jnp.minimum(w, 2)
    base = start_chunk * CHUNK

    pltpu.sync_copy(emb_hbm, table_v)
    pltpu.sync_copy(y_hbm.at[pl.ds(base, SPAN)], y_v)
    pltpu.sync_copy(m_hbm.at[pl.ds(base, SPAN)], m_v)

    def sel_body(i, carry):
        sl = pl.ds(i * LANES, LANES)
        idx_v[sl] = jnp.where(m_v[sl] != 0, y_v[sl], NUM_CLASSES)
        return carry
    lax.fori_loop(0, SPAN // LANES, sel_body, 0)

    rows = (rows0, rows1)
    wsem = (wsem0, wsem1)

    def write_desc(t, b):
        dst = out_hbm.at[pl.ds(base + t * CHUNK, CHUNK)]
        return pltpu.make_async_copy(rows[b], dst, wsem[b])

    def fill_chunk(t, b):
        def group_body(g, carry):
            idxv = idx_v[pl.ds(t * CHUNK + g * LANES, LANES)]
            for r in range(LANES):
                src_row = idxv[r]
                for c in range(VPR):
                    sl = pl.ds(c * LANES, LANES)
                    rows[b][g * LANES + r, sl] = table_v[src_row, sl]
            return carry
        lax.fori_loop(0, CHUNK // LANES, group_body, 0)

    def pair_body(j, carry):
        for h in range(2):
            t = 2 * j + h
            @pl.when(jnp.logical_and(t >= 2, t - 2 < nt))
            def _(t=t, b=h):
                write_desc(t - 2, b).wait()
            @pl.when(t < nt)
            def _(t=t, b=h):
                fill_chunk(t, b)
                write_desc(t, b).start()
        return carry
    lax.fori_loop(0, MAX_T // 2 + 1, pair_body, 0)


def kernel(y, mask, emb):
    y32 = jnp.pad(y.astype(jnp.int32), (0, N_PAD - N))
    m32 = jnp.pad(mask.astype(jnp.int32), (0, N_PAD - N))
    emb2 = jnp.concatenate(
        [emb, jnp.zeros((1, OUT_CHANNELS), emb.dtype)], axis=0)
    return _masked_gather(y32, m32, emb2)


# speedup vs baseline: 15.2921x; 2.4199x over previous
import functools

import jax
import jax.numpy as jnp
from jax import lax
from jax.experimental import pallas as pl
from jax.experimental.pallas import tpu as pltpu
from jax.experimental.pallas import tpu_sc as plsc

NUM_CLASSES = 40
OUT_CHANNELS = 512
N = 100000

NUM_WORKERS = 32
CHUNK = 80
NUM_CHUNKS = N // CHUNK
MAX_T = (NUM_CHUNKS + NUM_WORKERS - 1) // NUM_WORKERS
SPAN = MAX_T * CHUNK
N_PAD = (NUM_CHUNKS - 1) * CHUNK + SPAN
LANES = 16
TABLE_ROWS = 48


@functools.partial(
    pl.kernel,
    mesh=plsc.VectorSubcoreMesh(core_axis_name="c", subcore_axis_name="s"),
    out_type=jax.ShapeDtypeStruct((N, OUT_CHANNELS), jnp.float32),
    scratch_types=[
        pltpu.VMEM_SHARED((TABLE_ROWS, OUT_CHANNELS), jnp.float32),
        pltpu.VMEM((SPAN,), jnp.int32),
        pltpu.VMEM((SPAN,), jnp.int32),
        pltpu.VMEM((SPAN,), jnp.int32),
        pltpu.VMEM((2, 1, CHUNK), jnp.int32),
        pltpu.VMEM((CHUNK, OUT_CHANNELS), jnp.float32),
        pltpu.VMEM((CHUNK, OUT_CHANNELS), jnp.float32),
        pltpu.SemaphoreType.DMA,
        pltpu.SemaphoreType.DMA,
        pltpu.SemaphoreType.DMA,
        pltpu.SemaphoreType.DMA,
        pltpu.SemaphoreType.DMA,
    ],
)
def _masked_gather(y_hbm, m_hbm, emb_hbm, out_hbm,
                   table_s, y_v, m_v, idx_v, idxc, rows0, rows1,
                   gsem0, gsem1, wsem0, wsem1, tsem):
    w = lax.axis_index("s") * 2 + lax.axis_index("c")
    nt = jnp.where(w < 2, MAX_T, MAX_T - 1)
    start_chunk = (MAX_T - 1) * w + jnp.minimum(w, 2)
    base = start_chunk * CHUNK

    @pl.when(lax.axis_index("s") == 0)
    def _():
        pltpu.async_copy(emb_hbm, table_s, tsem).wait()
    pltpu.sync_copy(y_hbm.at[pl.ds(base, SPAN)], y_v)
    pltpu.sync_copy(m_hbm.at[pl.ds(base, SPAN)], m_v)

    def sel_body(i, carry):
        sl = pl.ds(i * LANES, LANES)
        idx_v[sl] = jnp.where(m_v[sl] != 0, y_v[sl], NUM_CLASSES)
        return carry
    lax.fori_loop(0, SPAN // LANES, sel_body, 0)
    plsc.subcore_barrier()

    rows = (rows0, rows1)
    gsem = (gsem0, gsem1)
    wsem = (wsem0, wsem1)

    def fill_chunk(t, b):
        def group_body(g, carry):
            idxv = idx_v[pl.ds(t * CHUNK + g * LANES, LANES)]
            for r in range(LANES):
                pltpu.make_async_copy(
                    table_s.at[idxv[r]], rows[b].at[g * LANES + r],
                    gsem[b]).start()
            return carry
        lax.fori_loop(0, CHUNK // LANES, group_body, 0)

    def drain_desc(t, b):
        return pltpu.make_async_copy(
            out_hbm.at[pl.ds(base + t * CHUNK, CHUNK)], rows[b], gsem[b])

    def write_desc(t, b):
        dst = out_hbm.at[pl.ds(base + t * CHUNK, CHUNK)]
        return pltpu.make_async_copy(rows[b], dst, wsem[b])

    def pair_body(j, carry):
        for h in range(2):
            t = 2 * j + h
            @pl.when(jnp.logical_and(t >= 2, t - 2 < nt))
            def _(t=t, b=h):
                write_desc(t - 2, b).wait()
            @pl.when(t < nt)
            def _(t=t, b=h):
                fill_chunk(t, b)
            @pl.when(jnp.logical_and(t >= 1, t - 1 < nt))
            def _(t=t, b=h):
                drain_desc(t - 1, 1 - b).wait()
                write_desc(t - 1, 1 - b).start()
        return carry
    lax.fori_loop(0, MAX_T // 2 + 1, pair_body, 0)


def kernel(y, mask, emb):
    y32 = jnp.pad(y.astype(jnp.int32), (0, N_PAD - N))
    m32 = jnp.pad(mask.astype(jnp.int32), (0, N_PAD - N))
    emb2 = jnp.concatenate(
        [emb, jnp.zeros((TABLE_ROWS - NUM_CLASSES, OUT_CHANNELS), emb.dtype)],
        axis=0)
    return _masked_gather(y32, m32, emb2)
